# baseline (device time: 130837 ns/iter reference)
import jax
import jax.numpy as jnp
from jax import lax
from jax.experimental import pallas as pl
from jax.experimental.pallas import tpu as pltpu

N_Z = 4
N_STR = 2


def kernel(x):
    m, n = x.shape
    qrows = m // 4
    chunk = qrows // N_Z
    sub = chunk // N_STR

    def body(x_ref, out_ref, comm_ref,
             rs_send, rs_recv, ag_send, ag_recv,
             xq_send, xq_recv, yq_send, yq_recv,
             xf_send, xf_recv, yf_send, yf_recv, dq_send, dq_recv):
        my_x = lax.axis_index("x")
        my_y = lax.axis_index("y")
        my_z = lax.axis_index("z")
        nxt = (my_z + 1) % N_Z
        h = my_y % 2
        partner = (1 - my_x, my_y, my_z)
        ypair = (my_x, jnp.bitwise_xor(my_y, 1), my_z)

        q_me = 2 * my_x + h
        q_xp = 2 * (1 - my_x) + h
        q_yh = 2 * my_x + (1 - h)

        def rows(q, c, g, size=sub):
            return pl.ds(q * qrows + c * chunk + g * sub, size)

        own = [(my_z + 1) % N_Z, my_z, (my_z - 1) % N_Z, (my_z - 2) % N_Z]

        out_ref[pl.ds(q_me * qrows, qrows), :] = x_ref[pl.ds(q_me * qrows, qrows), :]

        rs = [[None] * N_STR for _ in range(N_Z - 1)]

        def rs_start(s, g):
            send_c = (my_z - s) % N_Z
            rs[s][g] = pltpu.make_async_remote_copy(
                src_ref=out_ref.at[rows(q_me, send_c, g), :],
                dst_ref=comm_ref.at[s * N_STR + g],
                send_sem=rs_send.at[s * N_STR + g],
                recv_sem=rs_recv.at[s * N_STR + g],
                device_id=(my_x, my_y, nxt),
                device_id_type=pl.DeviceIdType.MESH,
            )
            rs[s][g].start()

        rs_start(0, 0)
        rs_start(0, 1)

        xq = [[None] * N_STR for _ in range(N_Z)]
        yq = [[None] * N_STR for _ in range(N_Z)]
        xf = [None] * N_Z
        yf = [None] * N_Z

        def publish(j, g):
            c = own[j]
            xq[j][g] = pltpu.make_async_remote_copy(
                src_ref=out_ref.at[rows(q_me, c, g), :],
                dst_ref=out_ref.at[rows(q_me, c, g), :],
                send_sem=xq_send.at[j * N_STR + g],
                recv_sem=xq_recv.at[j * N_STR + g],
                device_id=partner, device_id_type=pl.DeviceIdType.MESH,
            )
            xq[j][g].start()
            yq[j][g] = pltpu.make_async_remote_copy(
                src_ref=out_ref.at[rows(q_me, c, g), :],
                dst_ref=out_ref.at[rows(q_me, c, g), :],
                send_sem=yq_send.at[j * N_STR + g],
                recv_sem=yq_recv.at[j * N_STR + g],
                device_id=ypair, device_id_type=pl.DeviceIdType.MESH,
            )
            yq[j][g].start()

        ag = [[None] * N_STR for _ in range(N_Z - 1)]

        def ag_start(s, g):
            c = own[s]
            ag[s][g] = pltpu.make_async_remote_copy(
                src_ref=out_ref.at[rows(q_me, c, g), :],
                dst_ref=out_ref.at[rows(q_me, c, g), :],
                send_sem=ag_send.at[s * N_STR + g],
                recv_sem=ag_recv.at[s * N_STR + g],
                device_id=(my_x, my_y, nxt),
                device_id_type=pl.DeviceIdType.MESH,
            )
            ag[s][g].start()

        def forward(j):
            c = own[j]
            yq[j][0].wait_recv()
            xf[j] = pltpu.make_async_remote_copy(
                src_ref=out_ref.at[rows(q_yh, c, 0), :],
                dst_ref=out_ref.at[rows(q_yh, c, 0), :],
                send_sem=xf_send.at[j], recv_sem=xf_recv.at[j],
                device_id=partner, device_id_type=pl.DeviceIdType.MESH,
            )
            xf[j].start()
            xq[j][1].wait_recv()
            yf[j] = pltpu.make_async_remote_copy(
                src_ref=out_ref.at[rows(q_xp, c, 1), :],
                dst_ref=out_ref.at[rows(q_xp, c, 1), :],
                send_sem=yf_send.at[j], recv_sem=yf_recv.at[j],
                device_id=ypair, device_id_type=pl.DeviceIdType.MESH,
            )
            yf[j].start()

        for s in range(N_Z - 1):
            recv_c = (my_z - s - 1) % N_Z
            for g in range(N_STR):
                rs[s][g].wait()
                out_ref[rows(q_me, recv_c, g), :] += comm_ref[s * N_STR + g]
                if s < N_Z - 2:
                    rs_start(s + 1, g)
                else:
                    publish(0, g)
                    ag_start(0, g)

        for s in range(N_Z - 1):
            for g in range(N_STR):
                ag[s][g].wait()
                publish(s + 1, g)
                if s < N_Z - 2:
                    ag_start(s + 1, g)
            forward(s)

        dq = pltpu.make_async_remote_copy(
            src_ref=out_ref.at[rows(q_me, own[N_Z - 1], 0, chunk), :],
            dst_ref=out_ref.at[rows(q_me, own[N_Z - 1], 0, chunk), :],
            send_sem=dq_send.at[0], recv_sem=dq_recv.at[0],
            device_id=(1 - my_x, jnp.bitwise_xor(my_y, 1), my_z),
            device_id_type=pl.DeviceIdType.MESH,
        )
        dq.start()

        for j in range(N_Z - 1):
            xq[j][0].wait()
            xq[j][1].wait_send()
            yq[j][0].wait_send()
            yq[j][1].wait()
            xf[j].wait()
            yf[j].wait()
        for g in range(N_STR):
            xq[N_Z - 1][g].wait()
            yq[N_Z - 1][g].wait()
        dq.wait()

    return pl.pallas_call(
        body,
        out_shape=jax.ShapeDtypeStruct((m, n), x.dtype),
        in_specs=[pl.BlockSpec(memory_space=pltpu.VMEM)],
        out_specs=pl.BlockSpec(memory_space=pltpu.VMEM),
        scratch_shapes=[
            pltpu.VMEM(((N_Z - 1) * N_STR, sub, n), x.dtype),
            pltpu.SemaphoreType.DMA(((N_Z - 1) * N_STR,)),
            pltpu.SemaphoreType.DMA(((N_Z - 1) * N_STR,)),
            pltpu.SemaphoreType.DMA(((N_Z - 1) * N_STR,)),
            pltpu.SemaphoreType.DMA(((N_Z - 1) * N_STR,)),
            pltpu.SemaphoreType.DMA((N_Z * N_STR,)),
            pltpu.SemaphoreType.DMA((N_Z * N_STR,)),
            pltpu.SemaphoreType.DMA((N_Z * N_STR,)),
            pltpu.SemaphoreType.DMA((N_Z * N_STR,)),
            pltpu.SemaphoreType.DMA((N_Z,)),
            pltpu.SemaphoreType.DMA((N_Z,)),
            pltpu.SemaphoreType.DMA((N_Z,)),
            pltpu.SemaphoreType.DMA((N_Z,)),
            pltpu.SemaphoreType.DMA((1,)),
            pltpu.SemaphoreType.DMA((1,)),
        ],
    )(x)


# device time: 122625 ns/iter; 1.0670x vs baseline; 1.0670x over previous
import jax
import jax.numpy as jnp
from jax import lax
from jax.experimental import pallas as pl
from jax.experimental.pallas import tpu as pltpu

N_Z = 4
N_STR = 2


def kernel(x):
    m, n = x.shape
    qrows = m // 4
    chunk = qrows // N_Z
    sub = chunk // N_STR

    def body(x_ref, out_ref, comm_ref,
             rs_send, rs_recv, ag_send, ag_recv,
             xq_send, xq_recv, yq_send, yq_recv,
             xf_send, xf_recv, yf_send, yf_recv, dq_send, dq_recv):
        my_x = lax.axis_index("x")
        my_y = lax.axis_index("y")
        my_z = lax.axis_index("z")
        nxt = (my_z + 1) % N_Z
        h = my_y % 2
        partner = (1 - my_x, my_y, my_z)
        ypair = (my_x, jnp.bitwise_xor(my_y, 1), my_z)

        q_me = 2 * my_x + h
        q_xp = 2 * (1 - my_x) + h
        q_yh = 2 * my_x + (1 - h)

        diag = (1 - my_x, jnp.bitwise_xor(my_y, 1), my_z)
        barrier = pltpu.get_barrier_semaphore()
        for nbr in [(my_x, my_y, nxt), (my_x, my_y, (my_z - 1) % N_Z),
                    partner, ypair, diag]:
            pl.semaphore_signal(barrier, inc=1, device_id=nbr,
                                device_id_type=pl.DeviceIdType.MESH)
        pl.semaphore_wait(barrier, 5)

        def rows(q, c, g, size=sub):
            return pl.ds(q * qrows + c * chunk + g * sub, size)

        own = [(my_z + 1) % N_Z, my_z, (my_z - 1) % N_Z, (my_z - 2) % N_Z]

        out_ref[pl.ds(q_me * qrows, qrows), :] = x_ref[pl.ds(q_me * qrows, qrows), :]

        rs = [[None] * N_STR for _ in range(N_Z - 1)]

        def rs_start(s, g):
            send_c = (my_z - s) % N_Z
            rs[s][g] = pltpu.make_async_remote_copy(
                src_ref=out_ref.at[rows(q_me, send_c, g), :],
                dst_ref=comm_ref.at[s * N_STR + g],
                send_sem=rs_send.at[s * N_STR + g],
                recv_sem=rs_recv.at[s * N_STR + g],
                device_id=(my_x, my_y, nxt),
                device_id_type=pl.DeviceIdType.MESH,
            )
            rs[s][g].start()

        rs_start(0, 0)
        rs_start(0, 1)

        xq = [[None] * N_STR for _ in range(N_Z)]
        yq = [[None] * N_STR for _ in range(N_Z)]
        xf = [None] * N_Z
        yf = [None] * N_Z

        def publish(j, g):
            c = own[j]
            xq[j][g] = pltpu.make_async_remote_copy(
                src_ref=out_ref.at[rows(q_me, c, g), :],
                dst_ref=out_ref.at[rows(q_me, c, g), :],
                send_sem=xq_send.at[j * N_STR + g],
                recv_sem=xq_recv.at[j * N_STR + g],
                device_id=partner, device_id_type=pl.DeviceIdType.MESH,
            )
            xq[j][g].start()
            yq[j][g] = pltpu.make_async_remote_copy(
                src_ref=out_ref.at[rows(q_me, c, g), :],
                dst_ref=out_ref.at[rows(q_me, c, g), :],
                send_sem=yq_send.at[j * N_STR + g],
                recv_sem=yq_recv.at[j * N_STR + g],
                device_id=ypair, device_id_type=pl.DeviceIdType.MESH,
            )
            yq[j][g].start()

        ag = [[None] * N_STR for _ in range(N_Z - 1)]

        def ag_start(s, g):
            c = own[s]
            ag[s][g] = pltpu.make_async_remote_copy(
                src_ref=out_ref.at[rows(q_me, c, g), :],
                dst_ref=out_ref.at[rows(q_me, c, g), :],
                send_sem=ag_send.at[s * N_STR + g],
                recv_sem=ag_recv.at[s * N_STR + g],
                device_id=(my_x, my_y, nxt),
                device_id_type=pl.DeviceIdType.MESH,
            )
            ag[s][g].start()

        def forward(j):
            c = own[j]
            yq[j][0].wait_recv()
            xf[j] = pltpu.make_async_remote_copy(
                src_ref=out_ref.at[rows(q_yh, c, 0), :],
                dst_ref=out_ref.at[rows(q_yh, c, 0), :],
                send_sem=xf_send.at[j], recv_sem=xf_recv.at[j],
                device_id=partner, device_id_type=pl.DeviceIdType.MESH,
            )
            xf[j].start()
            xq[j][1].wait_recv()
            yf[j] = pltpu.make_async_remote_copy(
                src_ref=out_ref.at[rows(q_xp, c, 1), :],
                dst_ref=out_ref.at[rows(q_xp, c, 1), :],
                send_sem=yf_send.at[j], recv_sem=yf_recv.at[j],
                device_id=ypair, device_id_type=pl.DeviceIdType.MESH,
            )
            yf[j].start()

        for s in range(N_Z - 1):
            recv_c = (my_z - s - 1) % N_Z
            for g in range(N_STR):
                rs[s][g].wait()
                out_ref[rows(q_me, recv_c, g), :] += comm_ref[s * N_STR + g]
                if s < N_Z - 2:
                    rs_start(s + 1, g)
                else:
                    publish(0, g)
                    ag_start(0, g)

        for s in range(N_Z - 1):
            for g in range(N_STR):
                ag[s][g].wait()
                publish(s + 1, g)
                if s < N_Z - 2:
                    ag_start(s + 1, g)
            forward(s)

        dq = pltpu.make_async_remote_copy(
            src_ref=out_ref.at[rows(q_me, own[N_Z - 1], 0, chunk), :],
            dst_ref=out_ref.at[rows(q_me, own[N_Z - 1], 0, chunk), :],
            send_sem=dq_send.at[0], recv_sem=dq_recv.at[0],
            device_id=diag,
            device_id_type=pl.DeviceIdType.MESH,
        )
        dq.start()

        for j in range(N_Z - 1):
            xq[j][0].wait()
            xq[j][1].wait_send()
            yq[j][0].wait_send()
            yq[j][1].wait()
            xf[j].wait()
            yf[j].wait()
        for g in range(N_STR):
            xq[N_Z - 1][g].wait()
            yq[N_Z - 1][g].wait()
        dq.wait()

    return pl.pallas_call(
        body,
        out_shape=jax.ShapeDtypeStruct((m, n), x.dtype),
        in_specs=[pl.BlockSpec(memory_space=pltpu.VMEM)],
        out_specs=pl.BlockSpec(memory_space=pltpu.VMEM),
        scratch_shapes=[
            pltpu.VMEM(((N_Z - 1) * N_STR, sub, n), x.dtype),
            pltpu.SemaphoreType.DMA(((N_Z - 1) * N_STR,)),
            pltpu.SemaphoreType.DMA(((N_Z - 1) * N_STR,)),
            pltpu.SemaphoreType.DMA(((N_Z - 1) * N_STR,)),
            pltpu.SemaphoreType.DMA(((N_Z - 1) * N_STR,)),
            pltpu.SemaphoreType.DMA((N_Z * N_STR,)),
            pltpu.SemaphoreType.DMA((N_Z * N_STR,)),
            pltpu.SemaphoreType.DMA((N_Z * N_STR,)),
            pltpu.SemaphoreType.DMA((N_Z * N_STR,)),
            pltpu.SemaphoreType.DMA((N_Z,)),
            pltpu.SemaphoreType.DMA((N_Z,)),
            pltpu.SemaphoreType.DMA((N_Z,)),
            pltpu.SemaphoreType.DMA((N_Z,)),
            pltpu.SemaphoreType.DMA((1,)),
            pltpu.SemaphoreType.DMA((1,)),
        ],
        compiler_params=pltpu.CompilerParams(collective_id=0),
    )(x)


# device time: 122396 ns/iter; 1.0690x vs baseline; 1.0019x over previous
import jax
import jax.numpy as jnp
from jax import lax
from jax.experimental import pallas as pl
from jax.experimental.pallas import tpu as pltpu

N_Z = 4
N_STR = 2


def kernel(x):
    m, n = x.shape
    qrows = m // 4
    chunk = qrows // N_Z
    sub = chunk // N_STR

    def body(x_ref, out_ref, comm_ref,
             rs_send, rs_recv, ag_send, ag_recv,
             xq_send, xq_recv, yq_send, yq_recv,
             xf_send, xf_recv, yf_send, yf_recv, dq_send, dq_recv):
        my_x = lax.axis_index("x")
        my_y = lax.axis_index("y")
        my_z = lax.axis_index("z")
        nxt = (my_z + 1) % N_Z
        h = my_y % 2
        partner = (1 - my_x, my_y, my_z)
        ypair = (my_x, jnp.bitwise_xor(my_y, 1), my_z)

        q_me = 2 * my_x + h
        q_xp = 2 * (1 - my_x) + h
        q_yh = 2 * my_x + (1 - h)

        diag = (1 - my_x, jnp.bitwise_xor(my_y, 1), my_z)
        barrier = pltpu.get_barrier_semaphore()
        for nbr in [(my_x, my_y, nxt), (my_x, my_y, (my_z - 1) % N_Z),
                    partner, ypair, diag]:
            pl.semaphore_signal(barrier, inc=1, device_id=nbr,
                                device_id_type=pl.DeviceIdType.MESH)
        pl.semaphore_wait(barrier, 5)

        def rows(q, c, g, size=sub):
            return pl.ds(q * qrows + c * chunk + g * sub, size)

        own = [(my_z + 1) % N_Z, my_z, (my_z - 1) % N_Z, (my_z - 2) % N_Z]

        rs = [[None] * N_STR for _ in range(N_Z - 1)]

        def rs_start(s, g):
            send_c = (my_z - s) % N_Z
            src = x_ref if s == 0 else out_ref
            rs[s][g] = pltpu.make_async_remote_copy(
                src_ref=src.at[rows(q_me, send_c, g), :],
                dst_ref=comm_ref.at[s * N_STR + g],
                send_sem=rs_send.at[s * N_STR + g],
                recv_sem=rs_recv.at[s * N_STR + g],
                device_id=(my_x, my_y, nxt),
                device_id_type=pl.DeviceIdType.MESH,
            )
            rs[s][g].start()

        rs_start(0, 0)
        rs_start(0, 1)

        xq = [[None] * N_STR for _ in range(N_Z)]
        yq = [[None] * N_STR for _ in range(N_Z)]
        xf = [None] * N_Z
        yf = [None] * N_Z

        def publish(j, g):
            c = own[j]
            xq[j][g] = pltpu.make_async_remote_copy(
                src_ref=out_ref.at[rows(q_me, c, g), :],
                dst_ref=out_ref.at[rows(q_me, c, g), :],
                send_sem=xq_send.at[j * N_STR + g],
                recv_sem=xq_recv.at[j * N_STR + g],
                device_id=partner, device_id_type=pl.DeviceIdType.MESH,
            )
            xq[j][g].start()
            yq[j][g] = pltpu.make_async_remote_copy(
                src_ref=out_ref.at[rows(q_me, c, g), :],
                dst_ref=out_ref.at[rows(q_me, c, g), :],
                send_sem=yq_send.at[j * N_STR + g],
                recv_sem=yq_recv.at[j * N_STR + g],
                device_id=ypair, device_id_type=pl.DeviceIdType.MESH,
            )
            yq[j][g].start()

        ag = [[None] * N_STR for _ in range(N_Z - 1)]

        def ag_start(s, g):
            c = own[s]
            ag[s][g] = pltpu.make_async_remote_copy(
                src_ref=out_ref.at[rows(q_me, c, g), :],
                dst_ref=out_ref.at[rows(q_me, c, g), :],
                send_sem=ag_send.at[s * N_STR + g],
                recv_sem=ag_recv.at[s * N_STR + g],
                device_id=(my_x, my_y, nxt),
                device_id_type=pl.DeviceIdType.MESH,
            )
            ag[s][g].start()

        def forward(j):
            c = own[j]
            yq[j][0].wait_recv()
            xf[j] = pltpu.make_async_remote_copy(
                src_ref=out_ref.at[rows(q_yh, c, 0), :],
                dst_ref=out_ref.at[rows(q_yh, c, 0), :],
                send_sem=xf_send.at[j], recv_sem=xf_recv.at[j],
                device_id=partner, device_id_type=pl.DeviceIdType.MESH,
            )
            xf[j].start()
            xq[j][1].wait_recv()
            yf[j] = pltpu.make_async_remote_copy(
                src_ref=out_ref.at[rows(q_xp, c, 1), :],
                dst_ref=out_ref.at[rows(q_xp, c, 1), :],
                send_sem=yf_send.at[j], recv_sem=yf_recv.at[j],
                device_id=ypair, device_id_type=pl.DeviceIdType.MESH,
            )
            yf[j].start()

        for s in range(N_Z - 1):
            recv_c = (my_z - s - 1) % N_Z
            for g in range(N_STR):
                rs[s][g].wait()
                out_ref[rows(q_me, recv_c, g), :] = (
                    x_ref[rows(q_me, recv_c, g), :] + comm_ref[s * N_STR + g]
                )
                if s < N_Z - 2:
                    rs_start(s + 1, g)
                else:
                    publish(0, g)
                    ag_start(0, g)

        for s in range(N_Z - 1):
            for g in range(N_STR):
                ag[s][g].wait()
                publish(s + 1, g)
                if s < N_Z - 2:
                    ag_start(s + 1, g)
            forward(s)

        dq = pltpu.make_async_remote_copy(
            src_ref=out_ref.at[rows(q_me, own[N_Z - 1], 0, chunk), :],
            dst_ref=out_ref.at[rows(q_me, own[N_Z - 1], 0, chunk), :],
            send_sem=dq_send.at[0], recv_sem=dq_recv.at[0],
            device_id=diag,
            device_id_type=pl.DeviceIdType.MESH,
        )
        dq.start()

        for j in range(N_Z - 1):
            xq[j][0].wait()
            xq[j][1].wait_send()
            yq[j][0].wait_send()
            yq[j][1].wait()
            xf[j].wait()
            yf[j].wait()
        for g in range(N_STR):
            xq[N_Z - 1][g].wait()
            yq[N_Z - 1][g].wait()
        dq.wait()

    return pl.pallas_call(
        body,
        out_shape=jax.ShapeDtypeStruct((m, n), x.dtype),
        in_specs=[pl.BlockSpec(memory_space=pltpu.VMEM)],
        out_specs=pl.BlockSpec(memory_space=pltpu.VMEM),
        scratch_shapes=[
            pltpu.VMEM(((N_Z - 1) * N_STR, sub, n), x.dtype),
            pltpu.SemaphoreType.DMA(((N_Z - 1) * N_STR,)),
            pltpu.SemaphoreType.DMA(((N_Z - 1) * N_STR,)),
            pltpu.SemaphoreType.DMA(((N_Z - 1) * N_STR,)),
            pltpu.SemaphoreType.DMA(((N_Z - 1) * N_STR,)),
            pltpu.SemaphoreType.DMA((N_Z * N_STR,)),
            pltpu.SemaphoreType.DMA((N_Z * N_STR,)),
            pltpu.SemaphoreType.DMA((N_Z * N_STR,)),
            pltpu.SemaphoreType.DMA((N_Z * N_STR,)),
            pltpu.SemaphoreType.DMA((N_Z,)),
            pltpu.SemaphoreType.DMA((N_Z,)),
            pltpu.SemaphoreType.DMA((N_Z,)),
            pltpu.SemaphoreType.DMA((N_Z,)),
            pltpu.SemaphoreType.DMA((1,)),
            pltpu.SemaphoreType.DMA((1,)),
        ],
        compiler_params=pltpu.CompilerParams(collective_id=0),
    )(x)
